# baseline (speedup 1.0000x reference)
import functools

import jax
import jax.numpy as jnp
from jax.experimental import pallas as pl

_NUM_REPS = 512
_CODE_DIM = 32
_BLOCK_N = 8192


def _bf16_split3(x):
    xb = jax.lax.bitcast_convert_type(x, jnp.uint32)
    hi = jax.lax.bitcast_convert_type(xb & jnp.uint32(0xFFFF0000), jnp.float32)
    r = x - hi
    rb = jax.lax.bitcast_convert_type(r, jnp.uint32)
    mid = jax.lax.bitcast_convert_type(rb & jnp.uint32(0xFFFF0000), jnp.float32)
    return hi, mid, r - mid


def _cluster_block_kernel(embt_ref, cen_ref, rep_ref, loss_ref):
    i = pl.program_id(0)
    b = embt_ref.shape[1]
    et = embt_ref[:]
    cen = cen_ref[:]
    csq = jnp.sum(cen * cen, axis=1, keepdims=True)
    h, m, l = _bf16_split3(csq)
    lhs = jnp.concatenate([-2.0 * cen, h, m, l], axis=1)
    rhs = jnp.concatenate(
        [et, jnp.ones((3, b), jnp.float32)], axis=0)
    t = jax.lax.dot_general(
        lhs, rhs, (((1,), (0,)), ((), ())),
        preferred_element_type=jnp.float32)
    ngrp = _NUM_REPS // 8
    acc_v = t[0:8, :]
    acc_j = jnp.zeros((8, b), jnp.int32)
    for j in range(1, ngrp):
        v = t[8 * j:8 * j + 8, :]
        mask = v < acc_v
        acc_v = jnp.where(mask, v, acc_v)
        acc_j = jnp.where(mask, j, acc_j)
    md = jnp.min(acc_v, axis=0, keepdims=True)
    s_iota = jax.lax.broadcasted_iota(jnp.int32, acc_v.shape, 0)
    cand = jnp.where(acc_v == md, acc_j * 8 + s_iota, _NUM_REPS)
    rep = jnp.min(cand, axis=0)
    rep_ref[:] = rep[None, :]

    part = jnp.sum(et * et) + jnp.sum(md)

    @pl.when(i == 0)
    def _init():
        loss_ref[:, :] = jnp.zeros((1, 1), jnp.float32)

    loss_ref[:, :] += part.reshape(1, 1)


@functools.partial(jax.jit, static_argnums=())
def _cluster(embt, centers):
    n = embt.shape[1]
    grid = (n // _BLOCK_N,)
    rep2d, loss = pl.pallas_call(
        _cluster_block_kernel,
        grid=grid,
        in_specs=[
            pl.BlockSpec((_CODE_DIM, _BLOCK_N), lambda i: (0, i)),
            pl.BlockSpec((_NUM_REPS, _CODE_DIM), lambda i: (0, 0)),
        ],
        out_specs=[
            pl.BlockSpec((1, _BLOCK_N), lambda i: (0, i)),
            pl.BlockSpec((1, 1), lambda i: (0, 0)),
        ],
        out_shape=[
            jax.ShapeDtypeStruct((1, n), jnp.int32),
            jax.ShapeDtypeStruct((1, 1), jnp.float32),
        ],
    )(embt, centers)
    return rep2d, loss


def kernel(embs, centers):
    rep2d, loss = _cluster(embs.T, centers)
    return (centers, rep2d.reshape(embs.shape[0]), loss.reshape(()))


# speedup vs baseline: 1.0685x; 1.0685x over previous
import functools

import jax
import jax.numpy as jnp
from jax.experimental import pallas as pl

_NUM_REPS = 512
_CODE_DIM = 32
_BLOCK_N = 16384


def _bf16_split3(x):
    xb = jax.lax.bitcast_convert_type(x, jnp.uint32)
    hi = jax.lax.bitcast_convert_type(xb & jnp.uint32(0xFFFF0000), jnp.float32)
    r = x - hi
    rb = jax.lax.bitcast_convert_type(r, jnp.uint32)
    mid = jax.lax.bitcast_convert_type(rb & jnp.uint32(0xFFFF0000), jnp.float32)
    return hi, mid, r - mid


def _cluster_block_kernel(embt_ref, cen_ref, rep_ref, loss_ref):
    i = pl.program_id(0)
    b = embt_ref.shape[1]
    et = embt_ref[:]
    cen = cen_ref[:]
    csq = jnp.sum(cen * cen, axis=1, keepdims=True)
    h, m, l = _bf16_split3(csq)
    lhs = jnp.concatenate([-2.0 * cen, h, m, l], axis=1)
    rhs = jnp.concatenate(
        [et, jnp.ones((3, b), jnp.float32)], axis=0)
    t = jax.lax.dot_general(
        lhs, rhs, (((1,), (0,)), ((), ())),
        preferred_element_type=jnp.float32)
    ngrp = _NUM_REPS // 8
    acc_v = t[0:8, :]
    acc_j = jnp.zeros((8, b), jnp.int32)
    for j in range(1, ngrp):
        v = t[8 * j:8 * j + 8, :]
        mask = v < acc_v
        acc_v = jnp.where(mask, v, acc_v)
        acc_j = jnp.where(mask, j, acc_j)
    md = jnp.min(acc_v, axis=0, keepdims=True)
    s_iota = jax.lax.broadcasted_iota(jnp.int32, acc_v.shape, 0)
    cand = jnp.where(acc_v == md, acc_j * 8 + s_iota, _NUM_REPS)
    rep = jnp.min(cand, axis=0)
    rep_ref[:] = rep[None, :]

    part = jnp.sum(et * et) + jnp.sum(md)

    @pl.when(i == 0)
    def _init():
        loss_ref[:, :] = jnp.zeros((1, 1), jnp.float32)

    loss_ref[:, :] += part.reshape(1, 1)


@functools.partial(jax.jit, static_argnums=())
def _cluster(embt, centers):
    n = embt.shape[1]
    grid = (n // _BLOCK_N,)
    rep2d, loss = pl.pallas_call(
        _cluster_block_kernel,
        grid=grid,
        in_specs=[
            pl.BlockSpec((_CODE_DIM, _BLOCK_N), lambda i: (0, i)),
            pl.BlockSpec((_NUM_REPS, _CODE_DIM), lambda i: (0, 0)),
        ],
        out_specs=[
            pl.BlockSpec((1, _BLOCK_N), lambda i: (0, i)),
            pl.BlockSpec((1, 1), lambda i: (0, 0)),
        ],
        out_shape=[
            jax.ShapeDtypeStruct((1, n), jnp.int32),
            jax.ShapeDtypeStruct((1, 1), jnp.float32),
        ],
    )(embt, centers)
    return rep2d, loss


def kernel(embs, centers):
    rep2d, loss = _cluster(embs.T, centers)
    return (centers, rep2d.reshape(embs.shape[0]), loss.reshape(()))


# speedup vs baseline: 1.1019x; 1.0313x over previous
import functools

import jax
import jax.numpy as jnp
from jax.experimental import pallas as pl

_NUM_REPS = 512
_CODE_DIM = 32
_BLOCK_N = 32768
_K_CHUNKS = 2


def _bf16_split3(x):
    xb = jax.lax.bitcast_convert_type(x, jnp.uint32)
    hi = jax.lax.bitcast_convert_type(xb & jnp.uint32(0xFFFF0000), jnp.float32)
    r = x - hi
    rb = jax.lax.bitcast_convert_type(r, jnp.uint32)
    mid = jax.lax.bitcast_convert_type(rb & jnp.uint32(0xFFFF0000), jnp.float32)
    return hi, mid, r - mid


def _cluster_block_kernel(embt_ref, cen_ref, rep_ref, loss_ref):
    i = pl.program_id(0)
    b = embt_ref.shape[1]
    et = embt_ref[:]
    cen = cen_ref[:]
    csq = jnp.sum(cen * cen, axis=1, keepdims=True)
    h, m, l = _bf16_split3(csq)
    lhs = jnp.concatenate([-2.0 * cen, h, m, l], axis=1)
    rhs = jnp.concatenate(
        [et, jnp.ones((3, b), jnp.float32)], axis=0)
    kc = _NUM_REPS // _K_CHUNKS
    acc_v = None
    acc_j = jnp.zeros((8, b), jnp.int32)
    for c in range(_K_CHUNKS):
        t = jax.lax.dot_general(
            lhs[c * kc:(c + 1) * kc], rhs, (((1,), (0,)), ((), ())),
            preferred_element_type=jnp.float32)
        for jl in range(kc // 8):
            j = c * (kc // 8) + jl
            v = t[8 * jl:8 * jl + 8, :]
            if acc_v is None:
                acc_v = v
                continue
            mask = v < acc_v
            acc_v = jnp.where(mask, v, acc_v)
            acc_j = jnp.where(mask, j, acc_j)
    md = jnp.min(acc_v, axis=0, keepdims=True)
    s_iota = jax.lax.broadcasted_iota(jnp.int32, acc_v.shape, 0)
    cand = jnp.where(acc_v == md, acc_j * 8 + s_iota, _NUM_REPS)
    rep = jnp.min(cand, axis=0)
    rep_ref[:] = rep[None, :]

    part = jnp.sum(et * et) + jnp.sum(md)

    @pl.when(i == 0)
    def _init():
        loss_ref[:, :] = jnp.zeros((1, 1), jnp.float32)

    loss_ref[:, :] += part.reshape(1, 1)


@functools.partial(jax.jit, static_argnums=())
def _cluster(embt, centers):
    n = embt.shape[1]
    grid = (n // _BLOCK_N,)
    rep2d, loss = pl.pallas_call(
        _cluster_block_kernel,
        grid=grid,
        in_specs=[
            pl.BlockSpec((_CODE_DIM, _BLOCK_N), lambda i: (0, i)),
            pl.BlockSpec((_NUM_REPS, _CODE_DIM), lambda i: (0, 0)),
        ],
        out_specs=[
            pl.BlockSpec((1, _BLOCK_N), lambda i: (0, i)),
            pl.BlockSpec((1, 1), lambda i: (0, 0)),
        ],
        out_shape=[
            jax.ShapeDtypeStruct((1, n), jnp.int32),
            jax.ShapeDtypeStruct((1, 1), jnp.float32),
        ],
    )(embt, centers)
    return rep2d, loss


def kernel(embs, centers):
    rep2d, loss = _cluster(embs.T, centers)
    return (centers, rep2d.reshape(embs.shape[0]), loss.reshape(()))
